# baseline (speedup 1.0000x reference)
import functools

import jax
import jax.numpy as jnp
from jax import lax
from jax.experimental import pallas as pl
from jax.experimental.pallas import tpu as pltpu
from jax.experimental.pallas import tpu_sc as plsc

NW = 32
CB = 8
NBUF = 8


def _gates_body(ctx1_ref, ctx2_ref, w11_ref, b11_ref, w12_ref, b12_ref,
                w21_ref, b21_ref, w22_ref, b22_ref, g1_ref, g2_ref):
    h1 = jnp.maximum(
        jnp.dot(ctx1_ref[...], w11_ref[...],
                preferred_element_type=jnp.float32) + b11_ref[...], 0.0)
    g1_ref[...] = jax.nn.sigmoid(
        jnp.dot(h1, w12_ref[...],
                preferred_element_type=jnp.float32) + b12_ref[...]) * 2.0
    h2 = jnp.maximum(
        jnp.dot(ctx2_ref[...], w21_ref[...],
                preferred_element_type=jnp.float32) + b21_ref[...], 0.0)
    g2_ref[...] = jax.nn.sigmoid(
        jnp.dot(h2, w22_ref[...],
                preferred_element_type=jnp.float32) + b22_ref[...]) * 2.0


def _sc_body(x_hbm, g2_hbm, o2_hbm, g2buf, xb, gsem, insem, o2sem):
    B, L, D = x_hbm.shape
    nc = B // NW
    c_ax = lax.axis_index("c")
    s_ax = lax.axis_index("s")
    wid = s_ax * 2 + c_ax
    base = wid * nc

    pltpu.make_async_copy(g2_hbm, g2buf, gsem).start()
    pltpu.make_async_copy(g2_hbm, g2buf, gsem).wait()

    def in_copy(c, p):
        return pltpu.make_async_copy(
            x_hbm.at[base + c], xb.at[p], insem.at[p])

    def o2_copy(c, p):
        return pltpu.make_async_copy(
            xb.at[p], o2_hbm.at[base + c], o2sem.at[p])

    in_copy(0, 0).start()

    def step(c, carry):
        p = lax.rem(c, 2)

        @pl.when(c >= 1)
        def _():
            o2_copy(c - 1, 1 - p).wait()

        @pl.when(c + 1 < nc)
        def _():
            in_copy(c + 1, 1 - p).start()

        in_copy(c, p).wait()
        xp = xb.at[p]

        @plsc.parallel_loop(0, D // 16, step=1)
        def _(k):
            ds_k = pl.ds(k * 16, 16)
            g2v = g2buf[ds_k]
            for j in range(L):
                xp[j, ds_k] = xp[j, ds_k] * g2v

        o2_copy(c, p).start()
        return carry

    lax.fori_loop(0, nc, step, 0)
    o2_copy(nc - 1, lax.rem(nc - 1, 2)).wait()


def _tc_body(g1_ref, x_hbm, o1_hbm, xbuf, o1buf, in_sem, o1_sem):
    B = x_hbm.shape[0]
    nc = B // CB

    def in_copy(c, slot):
        return pltpu.make_async_copy(
            x_hbm.at[pl.ds(c * CB, CB)], xbuf.at[slot], in_sem.at[slot])

    def o1_copy(c, slot):
        return pltpu.make_async_copy(
            o1buf.at[slot], o1_hbm.at[pl.ds(c * CB, CB)], o1_sem.at[slot])

    for s in range(NBUF):
        in_copy(s, s).start(priority=s % 2)

    def round_body(r, carry):
        for s in range(NBUF):
            c = r * NBUF + s
            in_copy(c, s).wait()

            @pl.when(r >= 1)
            def _():
                o1_copy(c - NBUF, s).wait()

            o1buf[s] = xbuf[s] * g1_ref[...][None]
            o1_copy(c, s).start(priority=s % 2)

            @pl.when(c + NBUF < nc)
            def _():
                in_copy(c + NBUF, s).start(priority=(s + 1) % 2)

        return carry

    lax.fori_loop(0, nc // NBUF, round_body, 0)

    for s in range(NBUF):
        o1_copy(nc - NBUF + s, s).wait()


def kernel(feed_dict, flat_emb, fs1_ctx_bias, fs2_ctx_bias,
           fs1_W1, fs1_b1, fs1_W2, fs1_b2,
           fs2_W1, fs2_b1, fs2_W2, fs2_b2):
    B, L, D = flat_emb.shape
    E = fs1_ctx_bias.shape[-1]
    H = fs1_W1.shape[-1]

    g1, g2 = pl.pallas_call(
        _gates_body,
        out_shape=[
            jax.ShapeDtypeStruct((1, D), jnp.float32),
            jax.ShapeDtypeStruct((1, D), jnp.float32),
        ],
    )(fs1_ctx_bias, fs2_ctx_bias,
      fs1_W1, fs1_b1.reshape(1, H), fs1_W2, fs1_b2.reshape(1, D),
      fs2_W1, fs2_b1.reshape(1, H), fs2_W2, fs2_b2.reshape(1, D))

    mesh = plsc.VectorSubcoreMesh(core_axis_name="c", subcore_axis_name="s")
    sc = functools.partial(
        pl.kernel,
        mesh=mesh,
        out_type=jax.ShapeDtypeStruct((B, L, D), jnp.float32),
        scratch_types=[
            pltpu.VMEM((D,), jnp.float32),
            pltpu.VMEM((2, L, D), jnp.float32),
            pltpu.SemaphoreType.DMA,
            pltpu.SemaphoreType.DMA((2,)),
            pltpu.SemaphoreType.DMA((2,)),
        ],
    )(_sc_body)
    out2 = sc(flat_emb, g2.reshape(D))

    vmem = pl.BlockSpec(memory_space=pltpu.MemorySpace.VMEM)
    hbm = pl.BlockSpec(memory_space=pltpu.MemorySpace.HBM)
    out1 = pl.pallas_call(
        _tc_body,
        in_specs=[vmem, hbm],
        out_specs=hbm,
        out_shape=jax.ShapeDtypeStruct((B, L, D), jnp.float32),
        scratch_shapes=[
            pltpu.VMEM((NBUF, CB, L, D), jnp.float32),
            pltpu.VMEM((NBUF, CB, L, D), jnp.float32),
            pltpu.SemaphoreType.DMA((NBUF,)),
            pltpu.SemaphoreType.DMA((NBUF,)),
        ],
    )(g1, flat_emb)

    return (out1, out2)


# speedup vs baseline: 1.0461x; 1.0461x over previous
import functools

import jax
import jax.numpy as jnp
from jax import lax
from jax.experimental import pallas as pl
from jax.experimental.pallas import tpu as pltpu
from jax.experimental.pallas import tpu_sc as plsc

NW = 32


def _gates_body(ctx1_ref, ctx2_ref, w11_ref, b11_ref, w12_ref, b12_ref,
                w21_ref, b21_ref, w22_ref, b22_ref, g1_ref, g2_ref):
    h1 = jnp.maximum(
        jnp.dot(ctx1_ref[...], w11_ref[...],
                preferred_element_type=jnp.float32) + b11_ref[...], 0.0)
    g1_ref[...] = jax.nn.sigmoid(
        jnp.dot(h1, w12_ref[...],
                preferred_element_type=jnp.float32) + b12_ref[...]) * 2.0
    h2 = jnp.maximum(
        jnp.dot(ctx2_ref[...], w21_ref[...],
                preferred_element_type=jnp.float32) + b21_ref[...], 0.0)
    g2_ref[...] = jax.nn.sigmoid(
        jnp.dot(h2, w22_ref[...],
                preferred_element_type=jnp.float32) + b22_ref[...]) * 2.0


def _sc_body(x_hbm, g1_hbm, g2_hbm, o1_hbm, o2_hbm,
             g1buf, g2buf, xb, o1b, gsem, insem, o1sem, o2sem):
    B, L, D = x_hbm.shape
    nc = B // NW
    c_ax = lax.axis_index("c")
    s_ax = lax.axis_index("s")
    wid = s_ax * 2 + c_ax
    base = wid * nc

    pltpu.make_async_copy(g1_hbm, g1buf, gsem).start()
    pltpu.make_async_copy(g1_hbm, g1buf, gsem).wait()
    pltpu.make_async_copy(g2_hbm, g2buf, gsem).start()
    pltpu.make_async_copy(g2_hbm, g2buf, gsem).wait()

    def in_copy(c, p):
        return pltpu.make_async_copy(
            x_hbm.at[base + c], xb.at[p], insem.at[p])

    def o1_copy(c):
        return pltpu.make_async_copy(
            o1b, o1_hbm.at[base + c], o1sem)

    def o2_copy(c, p):
        return pltpu.make_async_copy(
            xb.at[p], o2_hbm.at[base + c], o2sem.at[p])

    in_copy(0, 0).start()

    def step(c, carry):
        p = lax.rem(c, 2)

        @pl.when(c >= 1)
        def _():
            o2_copy(c - 1, 1 - p).wait()

        @pl.when(c + 1 < nc)
        def _():
            in_copy(c + 1, 1 - p).start()

        in_copy(c, p).wait()

        @pl.when(c >= 1)
        def _():
            o1_copy(c - 1).wait()

        xp = xb.at[p]

        @plsc.parallel_loop(0, D // 16, step=1)
        def _(k):
            ds_k = pl.ds(k * 16, 16)
            g1v = g1buf[ds_k]
            for j in range(L):
                o1b[j, ds_k] = xp[j, ds_k] * g1v

        o1_copy(c).start()

        @plsc.parallel_loop(0, D // 16, step=1)
        def _(k):
            ds_k = pl.ds(k * 16, 16)
            g2v = g2buf[ds_k]
            for j in range(L):
                xp[j, ds_k] = xp[j, ds_k] * g2v

        o2_copy(c, p).start()

        return carry

    lax.fori_loop(0, nc, step, 0)

    o1_copy(nc - 1).wait()
    o2_copy(nc - 1, lax.rem(nc - 1, 2)).wait()


def kernel(feed_dict, flat_emb, fs1_ctx_bias, fs2_ctx_bias,
           fs1_W1, fs1_b1, fs1_W2, fs1_b2,
           fs2_W1, fs2_b1, fs2_W2, fs2_b2):
    B, L, D = flat_emb.shape
    E = fs1_ctx_bias.shape[-1]
    H = fs1_W1.shape[-1]

    g1, g2 = pl.pallas_call(
        _gates_body,
        out_shape=[
            jax.ShapeDtypeStruct((1, D), jnp.float32),
            jax.ShapeDtypeStruct((1, D), jnp.float32),
        ],
    )(fs1_ctx_bias, fs2_ctx_bias,
      fs1_W1, fs1_b1.reshape(1, H), fs1_W2, fs1_b2.reshape(1, D),
      fs2_W1, fs2_b1.reshape(1, H), fs2_W2, fs2_b2.reshape(1, D))
    g1 = g1.reshape(D)
    g2 = g2.reshape(D)

    mesh = plsc.VectorSubcoreMesh(core_axis_name="c", subcore_axis_name="s")
    sc = functools.partial(
        pl.kernel,
        mesh=mesh,
        out_type=[
            jax.ShapeDtypeStruct((B, L, D), jnp.float32),
            jax.ShapeDtypeStruct((B, L, D), jnp.float32),
        ],
        scratch_types=[
            pltpu.VMEM((D,), jnp.float32),
            pltpu.VMEM((D,), jnp.float32),
            pltpu.VMEM((2, L, D), jnp.float32),
            pltpu.VMEM((L, D), jnp.float32),
            pltpu.SemaphoreType.DMA,
            pltpu.SemaphoreType.DMA((2,)),
            pltpu.SemaphoreType.DMA,
            pltpu.SemaphoreType.DMA((2,)),
        ],
    )(_sc_body)
    out1, out2 = sc(flat_emb, g1, g2)
    return (out1, out2)
